# Initial kernel scaffold; baseline (speedup 1.0000x reference)
import jax
import jax.numpy as jnp
from jax.experimental import pallas as pl


def kernel(x, edge_index, edge_weight, W1, b1, W2, b2):
    raise NotImplementedError("write your pallas kernel here")


# speedup vs baseline: 7.2705x; 7.2705x over previous
import functools

import jax
import jax.numpy as jnp
from jax import lax
from jax.experimental import pallas as pl
from jax.experimental.pallas import tpu as pltpu
from jax.experimental.pallas import tpu_sc as plsc

NC = 2
NS = 16
LANES = 16
CHUNK = 80


def _deg_call(dst3, w3, n_pad):
    _, rows_pt, _ = dst3.shape
    zper = n_pad // NS

    mesh = plsc.VectorSubcoreMesh(core_axis_name="c", subcore_axis_name="s")

    @functools.partial(
        pl.kernel,
        out_type=jax.ShapeDtypeStruct((NC, n_pad), jnp.float32),
        mesh=mesh,
        compiler_params=pltpu.CompilerParams(use_tc_tiling_on_sc=False),
        scratch_types=[
            pltpu.VMEM((rows_pt, CHUNK), jnp.int32),
            pltpu.VMEM((rows_pt, CHUNK), jnp.float32),
            pltpu.VMEM((zper,), jnp.float32),
            pltpu.VMEM_SHARED((n_pad,), jnp.float32),
        ],
    )
    def k(dst_hbm, w_hbm, deg_hbm, dstb, wb, zb, deg_sh):
        c = lax.axis_index("c")
        s = lax.axis_index("s")
        wid = c * NS + s
        pltpu.sync_copy(dst_hbm.at[wid], dstb)
        pltpu.sync_copy(w_hbm.at[wid], wb)
        for i in range(zper // LANES):
            zb[pl.ds(i * LANES, LANES)] = jnp.zeros((LANES,), jnp.float32)
        pltpu.sync_copy(zb, deg_sh.at[pl.ds(s * zper, zper)])
        plsc.subcore_barrier()

        def body(g, _):
            pltpu.sync_copy(wb.at[g], deg_sh.at[dstb.at[g]], add=True)
            return 0

        lax.fori_loop(0, rows_pt, body, 0)
        plsc.subcore_barrier()

        @pl.when(s == 0)
        def _():
            pltpu.sync_copy(deg_sh, deg_hbm.at[c])

    return k(dst3, w3)


def _agg_call(hp, src3, dst3, w3):
    n, f = hp.shape
    _, rows_pt, _ = src3.shape
    zrows = 125
    zcopies = n // NS // zrows

    mesh = plsc.VectorSubcoreMesh(core_axis_name="c", subcore_axis_name="s")

    @functools.partial(
        pl.kernel,
        out_type=jax.ShapeDtypeStruct((NC, n, f), jnp.float32),
        mesh=mesh,
        compiler_params=pltpu.CompilerParams(use_tc_tiling_on_sc=False),
        scratch_types=[
            pltpu.VMEM((rows_pt, CHUNK), jnp.int32),
            pltpu.VMEM((rows_pt, CHUNK), jnp.int32),
            pltpu.VMEM((rows_pt, CHUNK), jnp.float32),
            pltpu.VMEM((CHUNK, f), jnp.float32),
            pltpu.VMEM((125, f), jnp.float32),
            pltpu.VMEM_SHARED((n, f), jnp.float32),
            pltpu.SemaphoreType.DMA,
        ],
    )
    def k(hp_hbm, src_hbm, dst_hbm, w_hbm, out_hbm,
          srcb, dstb, wb, rows, zb, out_sh, sem):
        c = lax.axis_index("c")
        s = lax.axis_index("s")
        wid = c * NS + s
        pltpu.sync_copy(src_hbm.at[wid], srcb)
        pltpu.sync_copy(dst_hbm.at[wid], dstb)
        pltpu.sync_copy(w_hbm.at[wid], wb)

        def zbody(i, _):
            for j in range(f // LANES):
                zb[i, pl.ds(j * LANES, LANES)] = jnp.zeros((LANES,), jnp.float32)
            return 0

        lax.fori_loop(0, zrows, zbody, 0)
        for r in range(zcopies):
            pltpu.sync_copy(
                zb, out_sh.at[pl.ds((s * zcopies + r) * zrows, zrows)])
        plsc.subcore_barrier()

        def body(g, _):
            pltpu.async_copy(hp_hbm.at[srcb.at[g]], rows, sem).wait()

            def scale(i16, _):
                wv = wb[g, pl.ds(i16 * LANES, LANES)]
                for r in range(LANES):
                    ws = wv[r]
                    row = i16 * LANES + r
                    for j in range(f // LANES):
                        sl = pl.ds(j * LANES, LANES)
                        rows[row, sl] = rows[row, sl] * ws
                return 0

            lax.fori_loop(0, CHUNK // LANES, scale, 0)
            pltpu.sync_copy(rows, out_sh.at[dstb.at[g]], add=True)
            return 0

        lax.fori_loop(0, rows_pt, body, 0)
        plsc.subcore_barrier()

        @pl.when(s == 0)
        def _():
            pltpu.sync_copy(out_sh, out_hbm.at[c])

    return k(hp, src3, dst3, w3)


def _norm_col(d0, d1):
    s = d0 + d1 + 1.0
    return jnp.where(s > 0, lax.rsqrt(s), 0.0)


def _layer1_tc(x, W1, deg0, deg1, rb):
    n, fin = x.shape
    h = W1.shape[1]
    hh = h // 2

    def body(x_r, w_r, d0_r, d1_r, ol_r, or_r):
        d = _norm_col(d0_r[...], d1_r[...])
        hp = d * jnp.dot(x_r[...], w_r[...],
                         preferred_element_type=jnp.float32)
        ol_r[...] = hp[:, :hh]
        or_r[...] = hp[:, hh:]

    return pl.pallas_call(
        body,
        grid=(n // rb,),
        in_specs=[
            pl.BlockSpec((rb, fin), lambda i: (i, 0)),
            pl.BlockSpec((fin, h), lambda i: (0, 0)),
            pl.BlockSpec((rb, 1), lambda i: (i, 0)),
            pl.BlockSpec((rb, 1), lambda i: (i, 0)),
        ],
        out_specs=[
            pl.BlockSpec((rb, hh), lambda i: (i, 0)),
            pl.BlockSpec((rb, hh), lambda i: (i, 0)),
        ],
        out_shape=[
            jax.ShapeDtypeStruct((n, hh), jnp.float32),
            jax.ShapeDtypeStruct((n, hh), jnp.float32),
        ],
    )(x, W1, deg0, deg1)


def _layer2_tc(aggs, hp1l, hp1r, deg0, deg1, b1, W2, rb):
    n, hh = hp1l.shape
    h = 2 * hh
    f2 = W2.shape[1]

    def body(al0, al1, ar0, ar1, hl, hr, d0_r, d1_r, b_r, w_r, ol_r, or_r):
        d = _norm_col(d0_r[...], d1_r[...])
        pre = jnp.concatenate(
            [al0[...] + al1[...] + hl[...], ar0[...] + ar1[...] + hr[...]],
            axis=1)
        t = jax.nn.relu(d * pre + b_r[...])
        hp2 = d * jnp.dot(t, w_r[...], preferred_element_type=jnp.float32)
        ol_r[...] = hp2[:, :f2 // 2]
        or_r[...] = hp2[:, f2 // 2:]

    half = pl.BlockSpec((rb, hh), lambda i: (i, 0))
    return pl.pallas_call(
        body,
        grid=(n // rb,),
        in_specs=[
            half, half, half, half, half, half,
            pl.BlockSpec((rb, 1), lambda i: (i, 0)),
            pl.BlockSpec((rb, 1), lambda i: (i, 0)),
            pl.BlockSpec((1, h), lambda i: (0, 0)),
            pl.BlockSpec((h, f2), lambda i: (0, 0)),
        ],
        out_specs=[
            pl.BlockSpec((rb, f2 // 2), lambda i: (i, 0)),
            pl.BlockSpec((rb, f2 // 2), lambda i: (i, 0)),
        ],
        out_shape=[
            jax.ShapeDtypeStruct((n, f2 // 2), jnp.float32),
            jax.ShapeDtypeStruct((n, f2 // 2), jnp.float32),
        ],
    )(*aggs, hp1l, hp1r, deg0, deg1, b1, W2)


def _final_tc(aggs, hp2l, hp2r, deg0, deg1, b2, rb):
    n, hh = hp2l.shape
    f2 = 2 * hh

    def body(al0, al1, ar0, ar1, hl, hr, d0_r, d1_r, b_r, o_r):
        d = _norm_col(d0_r[...], d1_r[...])
        pre = jnp.concatenate(
            [al0[...] + al1[...] + hl[...], ar0[...] + ar1[...] + hr[...]],
            axis=1)
        o_r[...] = d * pre + b_r[...]

    half = pl.BlockSpec((rb, hh), lambda i: (i, 0))
    return pl.pallas_call(
        body,
        grid=(n // rb,),
        in_specs=[
            half, half, half, half, half, half,
            pl.BlockSpec((rb, 1), lambda i: (i, 0)),
            pl.BlockSpec((rb, 1), lambda i: (i, 0)),
            pl.BlockSpec((1, f2), lambda i: (0, 0)),
        ],
        out_specs=pl.BlockSpec((rb, f2), lambda i: (i, 0)),
        out_shape=jax.ShapeDtypeStruct((n, f2), jnp.float32),
    )(*aggs, hp2l, hp2r, deg0, deg1, b2)


def kernel(x, edge_index, edge_weight, W1, b1, W2, b2):
    n = x.shape[0]
    src = edge_index[0].astype(jnp.int32)
    dst = edge_index[1].astype(jnp.int32)
    w = edge_weight.astype(jnp.float32)
    src3 = src.reshape(NC * NS, -1, CHUNK)
    dst3 = dst.reshape(NC * NS, -1, CHUNK)
    w3 = w.reshape(NC * NS, -1, CHUNK)
    n_pad = ((n + 1023) // 1024) * 1024

    deg = _deg_call(dst3, w3, n_pad)
    deg0 = deg[0, :n].reshape(n, 1)
    deg1 = deg[1, :n].reshape(n, 1)
    b1r = b1.reshape(1, -1)
    b2r = b2.reshape(1, -1)

    rb = 2000
    hp1l, hp1r = _layer1_tc(x, W1, deg0, deg1, rb)
    a1l = _agg_call(hp1l, src3, dst3, w3)
    a1r = _agg_call(hp1r, src3, dst3, w3)
    hp2l, hp2r = _layer2_tc(
        (a1l[0], a1l[1], a1r[0], a1r[1]),
        hp1l, hp1r, deg0, deg1, b1r, W2, rb)
    a2l = _agg_call(hp2l, src3, dst3, w3)
    a2r = _agg_call(hp2r, src3, dst3, w3)
    out = _final_tc(
        (a2l[0], a2l[1], a2r[0], a2r[1]),
        hp2l, hp2r, deg0, deg1, b2r, rb)
    return out


# speedup vs baseline: 9.4716x; 1.3028x over previous
import functools

import jax
import jax.numpy as jnp
from jax import lax
from jax.experimental import pallas as pl
from jax.experimental.pallas import tpu as pltpu
from jax.experimental.pallas import tpu_sc as plsc

NC = 2
NS = 16
LANES = 16
CHUNK = 80


def _deg_call(dst3, w3, n_pad):
    _, rows_pt, _ = dst3.shape
    zper = n_pad // NS

    mesh = plsc.VectorSubcoreMesh(core_axis_name="c", subcore_axis_name="s")

    @functools.partial(
        pl.kernel,
        out_type=jax.ShapeDtypeStruct((NC, n_pad), jnp.float32),
        mesh=mesh,
        compiler_params=pltpu.CompilerParams(use_tc_tiling_on_sc=False),
        scratch_types=[
            pltpu.VMEM((rows_pt, CHUNK), jnp.int32),
            pltpu.VMEM((rows_pt, CHUNK), jnp.float32),
            pltpu.VMEM((zper,), jnp.float32),
            pltpu.VMEM_SHARED((n_pad,), jnp.float32),
        ],
    )
    def k(dst_hbm, w_hbm, deg_hbm, dstb, wb, zb, deg_sh):
        c = lax.axis_index("c")
        s = lax.axis_index("s")
        wid = c * NS + s
        pltpu.sync_copy(dst_hbm.at[wid], dstb)
        pltpu.sync_copy(w_hbm.at[wid], wb)
        for i in range(zper // LANES):
            zb[pl.ds(i * LANES, LANES)] = jnp.zeros((LANES,), jnp.float32)
        pltpu.sync_copy(zb, deg_sh.at[pl.ds(s * zper, zper)])
        plsc.subcore_barrier()

        def body(g, _):
            pltpu.sync_copy(wb.at[g], deg_sh.at[dstb.at[g]], add=True)
            return 0

        lax.fori_loop(0, rows_pt, body, 0)
        plsc.subcore_barrier()

        @pl.when(s == 0)
        def _():
            pltpu.sync_copy(deg_sh, deg_hbm.at[c])

    return k(dst3, w3)


def _agg_call(hp, src3, dst3, w3):
    n, f = hp.shape
    _, rows_pt, _ = src3.shape
    zrows = 125
    zcopies = n // NS // zrows

    mesh = plsc.VectorSubcoreMesh(core_axis_name="c", subcore_axis_name="s")

    @functools.partial(
        pl.kernel,
        out_type=jax.ShapeDtypeStruct((NC, n, f), jnp.float32),
        mesh=mesh,
        compiler_params=pltpu.CompilerParams(use_tc_tiling_on_sc=False),
        scratch_types=[
            pltpu.VMEM((rows_pt, CHUNK), jnp.int32),
            pltpu.VMEM((rows_pt, CHUNK), jnp.int32),
            pltpu.VMEM((rows_pt, CHUNK), jnp.float32),
            pltpu.VMEM((CHUNK, f), jnp.float32),
            pltpu.VMEM((CHUNK, f), jnp.float32),
            pltpu.VMEM((125, f), jnp.float32),
            pltpu.VMEM_SHARED((n, f), jnp.float32),
            pltpu.SemaphoreType.DMA,
            pltpu.SemaphoreType.DMA,
            pltpu.SemaphoreType.DMA,
            pltpu.SemaphoreType.DMA,
        ],
    )
    def k(hp_hbm, src_hbm, dst_hbm, w_hbm, out_hbm,
          srcb, dstb, wb, rows0, rows1, zb, out_sh, gs0, gs1, ss0, ss1):
        c = lax.axis_index("c")
        s = lax.axis_index("s")
        wid = c * NS + s
        pltpu.sync_copy(src_hbm.at[wid], srcb)
        pltpu.sync_copy(dst_hbm.at[wid], dstb)
        pltpu.sync_copy(w_hbm.at[wid], wb)

        def start_gather(g, rows, sem):
            pltpu.async_copy(hp_hbm.at[srcb.at[g]], rows, sem)

        def wait_gather(g, rows, sem):
            pltpu.make_async_copy(hp_hbm.at[srcb.at[g]], rows, sem).wait()

        def start_scatter(g, rows, sem):
            pltpu.async_copy(rows, out_sh.at[dstb.at[g]], sem, add=True)

        def wait_scatter(g, rows, sem):
            pltpu.make_async_copy(rows, out_sh.at[dstb.at[g]], sem).wait()

        def scale(g, rows):
            def sbody(i16, _):
                wv = wb[g, pl.ds(i16 * LANES, LANES)]
                for r in range(LANES):
                    ws = wv[r]
                    row = i16 * LANES + r
                    for j in range(f // LANES):
                        sl = pl.ds(j * LANES, LANES)
                        rows[row, sl] = rows[row, sl] * ws
                return 0

            lax.fori_loop(0, CHUNK // LANES, sbody, 0)

        start_gather(0, rows0, gs0)

        def zbody(i, _):
            for j in range(f // LANES):
                zb[i, pl.ds(j * LANES, LANES)] = jnp.zeros((LANES,), jnp.float32)
            return 0

        lax.fori_loop(0, zrows, zbody, 0)
        for r in range(zcopies):
            pltpu.sync_copy(
                zb, out_sh.at[pl.ds((s * zcopies + r) * zrows, zrows)])
        plsc.subcore_barrier()

        def body(i, _):
            a = 2 * i
            b = a + 1
            start_gather(b, rows1, gs1)
            wait_gather(a, rows0, gs0)
            scale(a, rows0)
            start_scatter(a, rows0, ss0)
            wait_gather(b, rows1, gs1)
            scale(b, rows1)
            start_scatter(b, rows1, ss1)
            wait_scatter(a, rows0, ss0)
            start_gather(a + 2, rows0, gs0)
            wait_scatter(b, rows1, ss1)
            return 0

        last = rows_pt - 1
        lax.fori_loop(0, last // 2, body, 0)
        wait_gather(last, rows0, gs0)
        scale(last, rows0)
        start_scatter(last, rows0, ss0)
        wait_scatter(last, rows0, ss0)
        plsc.subcore_barrier()

        @pl.when(s == 0)
        def _():
            pltpu.sync_copy(out_sh, out_hbm.at[c])

    return k(hp, src3, dst3, w3)


def _norm_col(d0, d1):
    s = d0 + d1 + 1.0
    return jnp.where(s > 0, lax.rsqrt(s), 0.0)


def _layer1_tc(x, W1, deg0, deg1, rb):
    n, fin = x.shape
    h = W1.shape[1]
    hh = h // 2

    def body(x_r, w_r, d0_r, d1_r, ol_r, or_r):
        d = _norm_col(d0_r[...], d1_r[...])
        hp = d * jnp.dot(x_r[...], w_r[...],
                         preferred_element_type=jnp.float32)
        ol_r[...] = hp[:, :hh]
        or_r[...] = hp[:, hh:]

    return pl.pallas_call(
        body,
        grid=(n // rb,),
        in_specs=[
            pl.BlockSpec((rb, fin), lambda i: (i, 0)),
            pl.BlockSpec((fin, h), lambda i: (0, 0)),
            pl.BlockSpec((rb, 1), lambda i: (i, 0)),
            pl.BlockSpec((rb, 1), lambda i: (i, 0)),
        ],
        out_specs=[
            pl.BlockSpec((rb, hh), lambda i: (i, 0)),
            pl.BlockSpec((rb, hh), lambda i: (i, 0)),
        ],
        out_shape=[
            jax.ShapeDtypeStruct((n, hh), jnp.float32),
            jax.ShapeDtypeStruct((n, hh), jnp.float32),
        ],
    )(x, W1, deg0, deg1)


def _layer2_tc(aggs, hp1l, hp1r, deg0, deg1, b1, W2, rb):
    n, hh = hp1l.shape
    h = 2 * hh
    f2 = W2.shape[1]

    def body(al0, al1, ar0, ar1, hl, hr, d0_r, d1_r, b_r, w_r, ol_r, or_r):
        d = _norm_col(d0_r[...], d1_r[...])
        pre = jnp.concatenate(
            [al0[...] + al1[...] + hl[...], ar0[...] + ar1[...] + hr[...]],
            axis=1)
        t = jax.nn.relu(d * pre + b_r[...])
        hp2 = d * jnp.dot(t, w_r[...], preferred_element_type=jnp.float32)
        ol_r[...] = hp2[:, :f2 // 2]
        or_r[...] = hp2[:, f2 // 2:]

    half = pl.BlockSpec((rb, hh), lambda i: (i, 0))
    return pl.pallas_call(
        body,
        grid=(n // rb,),
        in_specs=[
            half, half, half, half, half, half,
            pl.BlockSpec((rb, 1), lambda i: (i, 0)),
            pl.BlockSpec((rb, 1), lambda i: (i, 0)),
            pl.BlockSpec((1, h), lambda i: (0, 0)),
            pl.BlockSpec((h, f2), lambda i: (0, 0)),
        ],
        out_specs=[
            pl.BlockSpec((rb, f2 // 2), lambda i: (i, 0)),
            pl.BlockSpec((rb, f2 // 2), lambda i: (i, 0)),
        ],
        out_shape=[
            jax.ShapeDtypeStruct((n, f2 // 2), jnp.float32),
            jax.ShapeDtypeStruct((n, f2 // 2), jnp.float32),
        ],
    )(*aggs, hp1l, hp1r, deg0, deg1, b1, W2)


def _final_tc(aggs, hp2l, hp2r, deg0, deg1, b2, rb):
    n, hh = hp2l.shape
    f2 = 2 * hh

    def body(al0, al1, ar0, ar1, hl, hr, d0_r, d1_r, b_r, o_r):
        d = _norm_col(d0_r[...], d1_r[...])
        pre = jnp.concatenate(
            [al0[...] + al1[...] + hl[...], ar0[...] + ar1[...] + hr[...]],
            axis=1)
        o_r[...] = d * pre + b_r[...]

    half = pl.BlockSpec((rb, hh), lambda i: (i, 0))
    return pl.pallas_call(
        body,
        grid=(n // rb,),
        in_specs=[
            half, half, half, half, half, half,
            pl.BlockSpec((rb, 1), lambda i: (i, 0)),
            pl.BlockSpec((rb, 1), lambda i: (i, 0)),
            pl.BlockSpec((1, f2), lambda i: (0, 0)),
        ],
        out_specs=pl.BlockSpec((rb, f2), lambda i: (i, 0)),
        out_shape=jax.ShapeDtypeStruct((n, f2), jnp.float32),
    )(*aggs, hp2l, hp2r, deg0, deg1, b2)


def kernel(x, edge_index, edge_weight, W1, b1, W2, b2):
    n = x.shape[0]
    src = edge_index[0].astype(jnp.int32)
    dst = edge_index[1].astype(jnp.int32)
    w = edge_weight.astype(jnp.float32)
    src3 = src.reshape(NC * NS, -1, CHUNK)
    dst3 = dst.reshape(NC * NS, -1, CHUNK)
    w3 = w.reshape(NC * NS, -1, CHUNK)
    n_pad = ((n + 1023) // 1024) * 1024

    deg = _deg_call(dst3, w3, n_pad)
    deg0 = deg[0, :n].reshape(n, 1)
    deg1 = deg[1, :n].reshape(n, 1)
    b1r = b1.reshape(1, -1)
    b2r = b2.reshape(1, -1)

    rb = 2000
    hp1l, hp1r = _layer1_tc(x, W1, deg0, deg1, rb)
    a1l = _agg_call(hp1l, src3, dst3, w3)
    a1r = _agg_call(hp1r, src3, dst3, w3)
    hp2l, hp2r = _layer2_tc(
        (a1l[0], a1l[1], a1r[0], a1r[1]),
        hp1l, hp1r, deg0, deg1, b1r, W2, rb)
    a2l = _agg_call(hp2l, src3, dst3, w3)
    a2r = _agg_call(hp2r, src3, dst3, w3)
    out = _final_tc(
        (a2l[0], a2l[1], a2r[0], a2r[1]),
        hp2l, hp2r, deg0, deg1, b2r, rb)
    return out


# speedup vs baseline: 11.6420x; 1.2291x over previous
import functools

import jax
import jax.numpy as jnp
from jax import lax
from jax.experimental import pallas as pl
from jax.experimental.pallas import tpu as pltpu
from jax.experimental.pallas import tpu_sc as plsc

NC = 2
NS = 16
LANES = 16
CHUNK = 80


def _deg_call(dst3, w3, n_pad):
    _, rows_pt, _ = dst3.shape
    zper = n_pad // NS

    mesh = plsc.VectorSubcoreMesh(core_axis_name="c", subcore_axis_name="s")

    @functools.partial(
        pl.kernel,
        out_type=jax.ShapeDtypeStruct((NC, n_pad), jnp.float32),
        mesh=mesh,
        compiler_params=pltpu.CompilerParams(use_tc_tiling_on_sc=False),
        scratch_types=[
            pltpu.VMEM((rows_pt, CHUNK), jnp.int32),
            pltpu.VMEM((rows_pt, CHUNK), jnp.float32),
            pltpu.VMEM((zper,), jnp.float32),
            pltpu.VMEM_SHARED((n_pad,), jnp.float32),
        ],
    )
    def k(dst_hbm, w_hbm, deg_hbm, dstb, wb, zb, deg_sh):
        c = lax.axis_index("c")
        s = lax.axis_index("s")
        wid = c * NS + s
        pltpu.sync_copy(dst_hbm.at[wid], dstb)
        pltpu.sync_copy(w_hbm.at[wid], wb)
        for i in range(zper // LANES):
            zb[pl.ds(i * LANES, LANES)] = jnp.zeros((LANES,), jnp.float32)
        pltpu.sync_copy(zb, deg_sh.at[pl.ds(s * zper, zper)])
        plsc.subcore_barrier()

        def body(g, _):
            pltpu.sync_copy(wb.at[g], deg_sh.at[dstb.at[g]], add=True)
            return 0

        lax.fori_loop(0, rows_pt, body, 0)
        plsc.subcore_barrier()

        @pl.when(s == 0)
        def _():
            pltpu.sync_copy(deg_sh, deg_hbm.at[c])

    return k(dst3, w3)


def _agg_call(hp, src3, dst3, w3):
    n, f = hp.shape
    _, rows_pt, _ = src3.shape
    zrows = 125
    zcopies = n // NS // zrows

    mesh = plsc.VectorSubcoreMesh(core_axis_name="c", subcore_axis_name="s")

    @functools.partial(
        pl.kernel,
        out_type=jax.ShapeDtypeStruct((NC, n, f), jnp.float32),
        mesh=mesh,
        compiler_params=pltpu.CompilerParams(use_tc_tiling_on_sc=False),
        scratch_types=[
            pltpu.VMEM((rows_pt, CHUNK), jnp.int32),
            pltpu.VMEM((rows_pt, CHUNK), jnp.int32),
            pltpu.VMEM((rows_pt, CHUNK), jnp.float32),
            pltpu.VMEM((CHUNK, f), jnp.float32),
            pltpu.VMEM((CHUNK, f), jnp.float32),
            pltpu.VMEM((CHUNK, f), jnp.float32),
            pltpu.VMEM((CHUNK, f), jnp.float32),
            pltpu.VMEM((125, f), jnp.float32),
            pltpu.VMEM_SHARED((n, f), jnp.float32),
            pltpu.SemaphoreType.DMA,
            pltpu.SemaphoreType.DMA,
            pltpu.SemaphoreType.DMA,
            pltpu.SemaphoreType.DMA,
            pltpu.SemaphoreType.DMA,
            pltpu.SemaphoreType.DMA,
            pltpu.SemaphoreType.DMA,
            pltpu.SemaphoreType.DMA,
        ],
    )
    def k(hp_hbm, src_hbm, dst_hbm, w_hbm, out_hbm,
          srcb, dstb, wb, rows0, rows1, rows2, rows3, zb, out_sh,
          gs0, gs1, gs2, gs3, ss0, ss1, ss2, ss3):
        c = lax.axis_index("c")
        s = lax.axis_index("s")
        wid = c * NS + s
        pltpu.sync_copy(src_hbm.at[wid], srcb)
        pltpu.sync_copy(dst_hbm.at[wid], dstb)
        pltpu.sync_copy(w_hbm.at[wid], wb)

        def start_gather(g, rows, sem):
            pltpu.async_copy(hp_hbm.at[srcb.at[g]], rows, sem)

        def wait_gather(g, rows, sem):
            pltpu.make_async_copy(hp_hbm.at[srcb.at[g]], rows, sem).wait()

        def start_scatter(g, rows, sem):
            pltpu.async_copy(rows, out_sh.at[dstb.at[g]], sem, add=True)

        def wait_scatter(g, rows, sem):
            pltpu.make_async_copy(rows, out_sh.at[dstb.at[g]], sem).wait()

        def scale(g, rows):
            def sbody(i16, _):
                wv = wb[g, pl.ds(i16 * LANES, LANES)]
                for r in range(LANES):
                    ws = wv[r]
                    row = i16 * LANES + r
                    for j in range(f // LANES):
                        sl = pl.ds(j * LANES, LANES)
                        rows[row, sl] = rows[row, sl] * ws
                return 0

            lax.fori_loop(0, CHUNK // LANES, sbody, 0)

        rowsl = (rows0, rows1, rows2, rows3)
        gsl = (gs0, gs1, gs2, gs3)
        ssl = (ss0, ss1, ss2, ss3)

        for g in range(3):
            start_gather(g, rowsl[g], gsl[g])

        def zbody(i, _):
            for j in range(f // LANES):
                zb[i, pl.ds(j * LANES, LANES)] = jnp.zeros((LANES,), jnp.float32)
            return 0

        lax.fori_loop(0, zrows, zbody, 0)
        for r in range(zcopies):
            pltpu.sync_copy(
                zb, out_sh.at[pl.ds((s * zcopies + r) * zrows, zrows)])
        plsc.subcore_barrier()

        def step(g, slot, prefetch):
            wait_gather(g, rowsl[slot], gsl[slot])
            scale(g, rowsl[slot])
            start_scatter(g, rowsl[slot], ssl[slot])
            if prefetch:
                pm1 = (slot + 3) % 4
                wait_scatter(g - 1, rowsl[pm1], ssl[pm1])
                start_gather(g + 3, rowsl[pm1], gsl[pm1])

        wait_gather(0, rows0, gs0)
        scale(0, rows0)
        start_scatter(0, rows0, ss0)
        start_gather(3, rows3, gs3)

        def body(i, _):
            for j in range(4):
                g = 4 * i + 1 + j
                step(g, (j + 1) % 4, True)
            return 0

        lax.fori_loop(0, 30, body, 0)
        step(121, 1, True)
        step(122, 2, False)
        step(123, 3, False)
        step(124, 0, False)
        for g, slot in ((121, 1), (122, 2), (123, 3), (124, 0)):
            wait_scatter(g, rowsl[slot], ssl[slot])
        plsc.subcore_barrier()

        @pl.when(s == 0)
        def _():
            pltpu.sync_copy(out_sh, out_hbm.at[c])

    return k(hp, src3, dst3, w3)


def _norm_col(d0, d1):
    s = d0 + d1 + 1.0
    return jnp.where(s > 0, lax.rsqrt(s), 0.0)


def _layer1_tc(x, W1, deg0, deg1, rb):
    n, fin = x.shape
    h = W1.shape[1]
    hh = h // 2

    def body(x_r, w_r, d0_r, d1_r, ol_r, or_r):
        d = _norm_col(d0_r[...], d1_r[...])
        hp = d * jnp.dot(x_r[...], w_r[...],
                         preferred_element_type=jnp.float32)
        ol_r[...] = hp[:, :hh]
        or_r[...] = hp[:, hh:]

    return pl.pallas_call(
        body,
        grid=(n // rb,),
        in_specs=[
            pl.BlockSpec((rb, fin), lambda i: (i, 0)),
            pl.BlockSpec((fin, h), lambda i: (0, 0)),
            pl.BlockSpec((rb, 1), lambda i: (i, 0)),
            pl.BlockSpec((rb, 1), lambda i: (i, 0)),
        ],
        out_specs=[
            pl.BlockSpec((rb, hh), lambda i: (i, 0)),
            pl.BlockSpec((rb, hh), lambda i: (i, 0)),
        ],
        out_shape=[
            jax.ShapeDtypeStruct((n, hh), jnp.float32),
            jax.ShapeDtypeStruct((n, hh), jnp.float32),
        ],
    )(x, W1, deg0, deg1)


def _layer2_tc(aggs, hp1l, hp1r, deg0, deg1, b1, W2, rb):
    n, hh = hp1l.shape
    h = 2 * hh
    f2 = W2.shape[1]

    def body(al0, al1, ar0, ar1, hl, hr, d0_r, d1_r, b_r, w_r, ol_r, or_r):
        d = _norm_col(d0_r[...], d1_r[...])
        pre = jnp.concatenate(
            [al0[...] + al1[...] + hl[...], ar0[...] + ar1[...] + hr[...]],
            axis=1)
        t = jax.nn.relu(d * pre + b_r[...])
        hp2 = d * jnp.dot(t, w_r[...], preferred_element_type=jnp.float32)
        ol_r[...] = hp2[:, :f2 // 2]
        or_r[...] = hp2[:, f2 // 2:]

    half = pl.BlockSpec((rb, hh), lambda i: (i, 0))
    return pl.pallas_call(
        body,
        grid=(n // rb,),
        in_specs=[
            half, half, half, half, half, half,
            pl.BlockSpec((rb, 1), lambda i: (i, 0)),
            pl.BlockSpec((rb, 1), lambda i: (i, 0)),
            pl.BlockSpec((1, h), lambda i: (0, 0)),
            pl.BlockSpec((h, f2), lambda i: (0, 0)),
        ],
        out_specs=[
            pl.BlockSpec((rb, f2 // 2), lambda i: (i, 0)),
            pl.BlockSpec((rb, f2 // 2), lambda i: (i, 0)),
        ],
        out_shape=[
            jax.ShapeDtypeStruct((n, f2 // 2), jnp.float32),
            jax.ShapeDtypeStruct((n, f2 // 2), jnp.float32),
        ],
    )(*aggs, hp1l, hp1r, deg0, deg1, b1, W2)


def _final_tc(aggs, hp2l, hp2r, deg0, deg1, b2, rb):
    n, hh = hp2l.shape
    f2 = 2 * hh

    def body(al0, al1, ar0, ar1, hl, hr, d0_r, d1_r, b_r, o_r):
        d = _norm_col(d0_r[...], d1_r[...])
        pre = jnp.concatenate(
            [al0[...] + al1[...] + hl[...], ar0[...] + ar1[...] + hr[...]],
            axis=1)
        o_r[...] = d * pre + b_r[...]

    half = pl.BlockSpec((rb, hh), lambda i: (i, 0))
    return pl.pallas_call(
        body,
        grid=(n // rb,),
        in_specs=[
            half, half, half, half, half, half,
            pl.BlockSpec((rb, 1), lambda i: (i, 0)),
            pl.BlockSpec((rb, 1), lambda i: (i, 0)),
            pl.BlockSpec((1, f2), lambda i: (0, 0)),
        ],
        out_specs=pl.BlockSpec((rb, f2), lambda i: (i, 0)),
        out_shape=jax.ShapeDtypeStruct((n, f2), jnp.float32),
    )(*aggs, hp2l, hp2r, deg0, deg1, b2)


def kernel(x, edge_index, edge_weight, W1, b1, W2, b2):
    n = x.shape[0]
    src = edge_index[0].astype(jnp.int32)
    dst = edge_index[1].astype(jnp.int32)
    w = edge_weight.astype(jnp.float32)
    src3 = src.reshape(NC * NS, -1, CHUNK)
    dst3 = dst.reshape(NC * NS, -1, CHUNK)
    w3 = w.reshape(NC * NS, -1, CHUNK)
    n_pad = ((n + 1023) // 1024) * 1024

    deg = _deg_call(dst3, w3, n_pad)
    deg0 = deg[0, :n].reshape(n, 1)
    deg1 = deg[1, :n].reshape(n, 1)
    b1r = b1.reshape(1, -1)
    b2r = b2.reshape(1, -1)

    rb = 2000
    hp1l, hp1r = _layer1_tc(x, W1, deg0, deg1, rb)
    a1l = _agg_call(hp1l, src3, dst3, w3)
    a1r = _agg_call(hp1r, src3, dst3, w3)
    hp2l, hp2r = _layer2_tc(
        (a1l[0], a1l[1], a1r[0], a1r[1]),
        hp1l, hp1r, deg0, deg1, b1r, W2, rb)
    a2l = _agg_call(hp2l, src3, dst3, w3)
    a2r = _agg_call(hp2r, src3, dst3, w3)
    out = _final_tc(
        (a2l[0], a2l[1], a2r[0], a2r[1]),
        hp2l, hp2r, deg0, deg1, b2r, rb)
    return out
